# Initial kernel scaffold; baseline (speedup 1.0000x reference)
import jax
import jax.numpy as jnp
from jax.experimental import pallas as pl


def kernel(x_nchw, w1, b1, w2, b2, w3, b3, w_lin, b_lin):
    raise NotImplementedError("write your pallas kernel here")


# speedup vs baseline: 10.7062x; 10.7062x over previous
import jax
import jax.numpy as jnp
from jax.experimental import pallas as pl
from jax.experimental.pallas import tpu as pltpu


BT1 = 64
BT2 = 32
CH2 = 512
CH3 = 512


def _parallel(n):
    return pltpu.CompilerParams(dimension_semantics=("parallel",) * n)


def _stage1_kernel(xe_ref, xo_ref, we_ref, wo_ref, b_ref, o_ref):
    n = BT1 * 16
    xe = xe_ref[...].reshape(n, 96)
    xo = xo_ref[...].reshape(n, 96)
    bias = b_ref[...]
    we = we_ref[...]
    wo = wo_ref[...]
    zero_row = jnp.zeros((16, 1, 448), jnp.bfloat16)
    for start in range(0, n, 256):
        a = xe[start:start + 256, :]
        b = xo[start:start + 256, :]
        m = jnp.maximum(
            jnp.maximum(
                jnp.dot(a, we, preferred_element_type=jnp.float32),
                jnp.dot(a, wo, preferred_element_type=jnp.float32)),
            jnp.maximum(
                jnp.dot(b, we, preferred_element_type=jnp.float32),
                jnp.dot(b, wo, preferred_element_type=jnp.float32)))
        pooled = jnp.maximum(m + bias, 0.0).astype(jnp.bfloat16)
        blk = pooled.reshape(16, 16, 448)
        i0 = start // 16
        o_ref[pl.ds(i0, 16), 0:1, :] = zero_row
        o_ref[pl.ds(i0, 16), 1:15, :] = blk[:, 0:14, :]
        o_ref[pl.ds(i0, 16), 15:16, :] = zero_row


def _stage2_kernel(x_ref, w2_ref, b2_ref, w3_ref, b3_ref, wl_ref, bl_ref,
                   o_ref, cat2, y2, cm2, s3, y3, cm3, p3):
    n2 = BT2 * 256
    xv = x_ref[...]
    cat2[:, 0:32] = xv
    cat2[pl.ds(0, n2 - 1), 32:64] = xv[1:, :]
    cat2[pl.ds(0, n2 - 2), 64:96] = xv[2:, :]
    b2 = b2_ref[...]
    end2 = n2 - 34
    for start in range(0, end2, CH2):
        size = min(CH2, end2 - start)
        acc = jnp.zeros((size, 128), jnp.float32)
        for kh in range(3):
            lhs = cat2[pl.ds(start + 16 * kh, size), :]
            acc = acc + jnp.dot(lhs, w2_ref[pl.ds(96 * kh, 96), :],
                                preferred_element_type=jnp.float32)
        y2[pl.ds(start, size), :] = jnp.maximum(acc + b2, 0.0)
    cme = y2[pl.ds(0, BT2 * 128, 2), :]
    cmo = y2[pl.ds(1, BT2 * 128, 2), :]
    cm2[...] = jnp.maximum(cme, cmo).reshape(BT2, 128, 128)
    s3[...] = jnp.zeros((BT2, 96, 128), jnp.bfloat16)
    for q in range(7):
        pa = cm2[:, pl.ds(16 * q, 8), :]
        pb = cm2[:, pl.ds(16 * q + 8, 8), :]
        blk = jnp.maximum(pa, pb)[:, 0:7, :].astype(jnp.bfloat16)
        s3[:, pl.ds((q + 1) * 10 + 1, 7), :] = blk
    n3 = BT2 * 96
    b3 = b3_ref[...]
    end3 = n3 - 29
    s3f = s3[...].reshape(n3, 128)
    for start in range(0, end3, CH3):
        size = min(CH3, end3 - start)
        acc = jnp.zeros((size, 128), jnp.float32)
        for kh in range(3):
            for kw in range(3):
                lhs = s3f[start + 10 * kh + kw:start + 10 * kh + kw + size, :]
                k = kh * 3 + kw
                acc = acc + jnp.dot(lhs, w3_ref[pl.ds(128 * k, 128), :],
                                    preferred_element_type=jnp.float32)
        y3[pl.ds(start, size), :] = jnp.maximum(acc + b3, 0.0)
    c3e = y3[pl.ds(0, BT2 * 48, 2), :]
    c3o = y3[pl.ds(1, BT2 * 48, 2), :]
    cm3[...] = jnp.maximum(c3e, c3o).reshape(BT2, 48, 128)
    for q in range(3):
        pa = cm3[:, pl.ds(10 * q, 5), :]
        pb = cm3[:, pl.ds(10 * q + 5, 5), :]
        p3[:, pl.ds(5 * q, 5), :] = jnp.maximum(pa, pb)
    acc = jnp.broadcast_to(bl_ref[...], (BT2, 128)).astype(jnp.float32)
    for q in range(3):
        for e in range(3):
            lhs = p3[:, q * 5 + e, :].astype(jnp.bfloat16)
            j = q * 3 + e
            acc = acc + jnp.dot(lhs, wl_ref[pl.ds(128 * j, 128), :],
                                preferred_element_type=jnp.float32)
    o_ref[...] = acc


def _build_banded_w1(w1, parity):
    w14 = w1.astype(jnp.float32).reshape(3, 3, 32)
    j = jnp.arange(32)[:, None]
    e = jnp.arange(14)[None, :]
    kwi = j - 2 * e - parity
    valid = (kwi >= 0) & (kwi <= 2)
    kwc = jnp.clip(kwi, 0, 2)
    w4 = jnp.take(w14, kwc, axis=1)
    w4 = jnp.where(valid[None, :, :, None], w4, 0.0)
    return w4.reshape(96, 448).astype(jnp.bfloat16)


def kernel(x_nchw, w1, b1, w2, b2, w3, b3, w_lin, b_lin):
    B = x_nchw.shape[0]
    x = x_nchw.reshape(B, 28, 28).astype(jnp.bfloat16)
    xpad = jnp.pad(x, ((0, 0), (1, 2), (1, 3)))
    xk = jnp.concatenate(
        [xpad[:, 0:28, :], xpad[:, 1:29, :], xpad[:, 2:30, :]], axis=-1)
    xk_e = jnp.pad(xk[:, 0:28:2, :], ((0, 0), (0, 2), (0, 0)))
    xk_o = jnp.pad(xk[:, 1:28:2, :], ((0, 0), (0, 2), (0, 0)))

    we = _build_banded_w1(w1, 0)
    wo = _build_banded_w1(w1, 1)
    b1t = jnp.tile(b1.astype(jnp.float32), (1, 14))

    out1 = pl.pallas_call(
        _stage1_kernel,
        out_shape=jax.ShapeDtypeStruct((B, 16, 448), jnp.bfloat16),
        grid=(B // BT1,),
        in_specs=[
            pl.BlockSpec((BT1, 16, 96), lambda i: (i, 0, 0)),
            pl.BlockSpec((BT1, 16, 96), lambda i: (i, 0, 0)),
            pl.BlockSpec((96, 448), lambda i: (0, 0)),
            pl.BlockSpec((96, 448), lambda i: (0, 0)),
            pl.BlockSpec((1, 448), lambda i: (0, 0)),
        ],
        out_specs=pl.BlockSpec((BT1, 16, 448), lambda i: (i, 0, 0)),
        compiler_params=_parallel(1),
        cost_estimate=pl.CostEstimate(
            flops=2 * B * 32 * 96 * 896, transcendentals=0,
            bytes_accessed=B * (32 * 96 + 16 * 448) * 2),
    )(xk_e, xk_o, we, wo, b1t)

    strip2 = jnp.pad(out1.reshape(B, 16, 14, 32),
                     ((0, 0), (0, 0), (1, 1), (0, 0)))
    strip2 = strip2.reshape(B * 256, 32)

    logits = pl.pallas_call(
        _stage2_kernel,
        out_shape=jax.ShapeDtypeStruct((B, 128), jnp.float32),
        grid=(B // BT2,),
        in_specs=[
            pl.BlockSpec((BT2 * 256, 32), lambda i: (i, 0)),
            pl.BlockSpec((288, 128), lambda i: (0, 0)),
            pl.BlockSpec((1, 128), lambda i: (0, 0)),
            pl.BlockSpec((1152, 128), lambda i: (0, 0)),
            pl.BlockSpec((1, 128), lambda i: (0, 0)),
            pl.BlockSpec((1152, 128), lambda i: (0, 0)),
            pl.BlockSpec((1, 128), lambda i: (0, 0)),
        ],
        out_specs=pl.BlockSpec((BT2, 128), lambda i: (i, 0)),
        scratch_shapes=[
            pltpu.VMEM((BT2 * 256, 96), jnp.bfloat16),
            pltpu.VMEM((BT2 * 256, 128), jnp.float32),
            pltpu.VMEM((BT2, 128, 128), jnp.float32),
            pltpu.VMEM((BT2, 96, 128), jnp.bfloat16),
            pltpu.VMEM((BT2 * 96, 128), jnp.float32),
            pltpu.VMEM((BT2, 48, 128), jnp.float32),
            pltpu.VMEM((BT2, 15, 128), jnp.float32),
        ],
        compiler_params=_parallel(1),
        cost_estimate=pl.CostEstimate(
            flops=2 * B * (224 * 288 * 128 + 70 * 1152 * 128 + 1152 * 128),
            transcendentals=0,
            bytes_accessed=B * (256 * 32 * 2 + 128 * 4)),
    )(strip2, w2, b2, w3, b3, w_lin, b_lin)

    return logits[:, :10]
